# Initial kernel scaffold; baseline (speedup 1.0000x reference)
import jax
import jax.numpy as jnp
from jax.experimental import pallas as pl


def kernel(fake_coarse_alpha, fake_coarse_error, fake_coarse_hidden, input_tensor, conv1_w, conv1_b, conv2_w, conv2_b, conv3_w, conv3_b, conv4_w, conv4_b):
    raise NotImplementedError("write your pallas kernel here")


# speedup vs baseline: 27.6822x; 27.6822x over previous
import functools

import jax
import jax.numpy as jnp
from jax import lax
from jax.experimental import pallas as pl
from jax.experimental.pallas import tpu as pltpu
from jax.experimental.pallas import tpu_sc as plsc

B = 8
HC = WC = 128
H = W = 512
KSEL = 1024
NPAT = B * KSEL
NC, NS = 2, 16
NW = NC * NS
PPT = NPAT // NW
NP = 64
LIM = W // 2 - 8

_i32 = jnp.int32
_f32 = jnp.float32
_bf16 = jnp.bfloat16


def _sortkey(x):
    b = lax.bitcast_convert_type(x + 0.0, _i32)
    return jnp.where(b >= 0, b, jnp.bitwise_xor(~b, _i32(-2147483648)))


def _k1_body(err1_ref, err2_ref, g1_ref, s_ref):
    key_all = _sortkey(err1_ref[...])

    t = jnp.full((B, 1), -2147483648, _i32)
    for bit in range(31, -1, -1):
        if bit == 31:
            t2 = jnp.zeros((B, 1), _i32)
        else:
            t2 = t + _i32(1 << bit)
        cnt = jnp.sum((key_all >= t2).astype(_i32), axis=1, keepdims=True)
        t = jnp.where(cnt >= KSEL, t2, t)

    rw = lax.broadcasted_iota(_i32, (128, 128), 0)
    cl = lax.broadcasted_iota(_i32, (128, 128), 1)
    U = (rw <= cl).astype(_bf16)
    Ls = (cl < rw).astype(_bf16)
    jh8 = lax.broadcasted_iota(_i32, (8, 1), 0)
    jl128 = lax.broadcasted_iota(_i32, (1, 128), 1)
    o32 = lax.broadcasted_iota(_i32, (32, 1), 0)
    offs1 = (o32 >> 2) * 128 + (o32 & 3)
    o16 = lax.broadcasted_iota(_i32, (16, 1), 0)
    offs3 = (o16 >> 2) * 512 + (o16 & 3)

    def rowmajor_cumsum(Mb):
        rowc = jnp.dot(Mb, U, preferred_element_type=_f32)
        pref = jnp.dot(Ls, rowc[:, 127:128].astype(_bf16),
                       preferred_element_type=_f32)
        return rowc + pref

    for b in range(B):
        D = err2_ref[b * 128:(b + 1) * 128, :]
        kb = _sortkey(D)
        tb = t[b:b + 1, :]
        gt = kb > tb
        eq = kb == tb
        k1 = jnp.sum(gt.astype(_i32), keepdims=True).reshape(1, 1)
        quota = (KSEL - k1).astype(_f32)
        eqrank = rowmajor_cumsum(eq.astype(_bf16))
        sel = gt | (eq & (eqrank <= quota))
        psum = rowmajor_cumsum(sel.astype(_bf16))

        g = psum.astype(_i32) >> 7
        m = psum.astype(_i32) & 127
        t1 = [jnp.sum((g < jh).astype(_f32), keepdims=True).reshape(1, 1)
              for jh in range(8)]
        term1 = jnp.concatenate(t1, axis=0)
        gT = jnp.transpose(g)
        mT = jnp.transpose(m)
        acc = jnp.zeros((8, 128), _f32)
        for r0 in range(0, 128, 8):
            apc = [(g[r0 + rr:r0 + rr + 1, :] == jh8).astype(_bf16)
                   for rr in range(8)]
            A2 = jnp.concatenate(apc, axis=1)
            bpc = [((mT[:, r0 + rr:r0 + rr + 1] <= jl128) &
                    (gT[:, r0 + rr:r0 + rr + 1] >= 0)).astype(_bf16)
                   for rr in range(8)]
            B2 = jnp.concatenate(bpc, axis=0)
            acc = acc + jnp.dot(A2, B2, preferred_element_type=_f32)
        sel_idx = (term1 + acc).astype(_i32)

        hh = sel_idx >> 7
        ww = sel_idx & 127
        tlh = jnp.clip(hh * 2 - 4, 0, LIM)
        tlw = jnp.clip(ww * 2 - 4, 0, LIM)
        base1 = _i32(b * 32768) + tlh * 128 + (tlw >> 1)
        base3 = _i32(b * 262144) + (tlh * 2 + 2) * 512 + tlw * 2 + 2
        for jh in range(8):
            col = b * 1024 + jh * 128
            g1_ref[:, col:col + 128] = offs1 + base1[jh:jh + 1, :]
            s_ref[:, col:col + 128] = offs3 + base3[jh:jh + 1, :]


def _topk_indices(err1, err2):
    return pl.pallas_call(
        _k1_body,
        out_shape=(jax.ShapeDtypeStruct((32, NPAT), _i32),
                   jax.ShapeDtypeStruct((16, NPAT), _i32)),
    )(err1, err2)


def _im2col(x, osz):
    cols = [x[:, dy:dy + osz, dx:dx + osz, :]
            for dy in range(3) for dx in range(3)]
    c = jnp.concatenate(cols, axis=-1)
    return c.reshape(x.shape[0] * osz * osz, c.shape[-1])


def _k3_body(sp_ref, w1_ref, b1_ref, w2_ref, b2_ref, w3_ref, b3_ref,
             w4_ref, b4_ref, out_ref):
    sp = sp_ref[...].astype(_bf16)
    s = sp[..., 0:36]
    q = sp[:, 0:4, 0:4, :]
    ms = [[q[..., 36 + dh * 6 + dw * 3:39 + dh * 6 + dw * 3] for dw in (0, 1)]
          for dh in (0, 1)]
    aws = [jnp.concatenate(
        sum(([ms[dh][0][:, :, c:c + 1, :], ms[dh][1][:, :, c:c + 1, :]]
             for c in range(4)), []), axis=2) for dh in (0, 1)]
    mp = jnp.concatenate(
        sum(([aws[0][:, r:r + 1], aws[1][:, r:r + 1]] for r in range(4)), []),
        axis=1)
    x1 = jnp.dot(_im2col(s, 6), w1_ref[...], preferred_element_type=_f32)
    x1 = jnp.maximum(x1 + b1_ref[...], 0.0).reshape(NP, 6, 6, 32).astype(_bf16)
    x2 = jnp.dot(_im2col(x1, 4), w2_ref[...], preferred_element_type=_f32)
    x2 = jnp.maximum(x2 + b2_ref[...], 0.0).reshape(NP, 4, 4, 64).astype(_bf16)
    xr = jnp.concatenate([x2[:, i // 2:i // 2 + 1, :, :] for i in range(8)],
                         axis=1)
    xu = jnp.concatenate([xr[:, :, i // 2:i // 2 + 1, :] for i in range(8)],
                         axis=2)
    mid = jnp.concatenate([xu, mp], axis=-1)
    x3 = jnp.dot(_im2col(mid, 6), w3_ref[...], preferred_element_type=_f32)
    x3 = jnp.maximum(x3 + b3_ref[...], 0.0).reshape(NP, 6, 6, 32).astype(_bf16)
    x4 = jnp.dot(_im2col(x3, 4), w4_ref[...], preferred_element_type=_f32)
    out_ref[...] = x4 + b4_ref[...]


def _convs(startP, w1, b1, w2, b2, w3, b3, w4, b4):
    nblk = NPAT // NP
    return pl.pallas_call(
        _k3_body,
        grid=(nblk,),
        in_specs=[
            pl.BlockSpec((NP, 8, 8, 64), lambda i: (i, 0, 0, 0)),
            pl.BlockSpec((324, 32), lambda i: (0, 0)),
            pl.BlockSpec((1, 32), lambda i: (0, 0)),
            pl.BlockSpec((288, 64), lambda i: (0, 0)),
            pl.BlockSpec((1, 64), lambda i: (0, 0)),
            pl.BlockSpec((603, 32), lambda i: (0, 0)),
            pl.BlockSpec((1, 32), lambda i: (0, 0)),
            pl.BlockSpec((288, 8), lambda i: (0, 0)),
            pl.BlockSpec((1, 8), lambda i: (0, 0)),
        ],
        out_specs=pl.BlockSpec((NP * 16, 8), lambda i: (i, 0)),
        out_shape=jax.ShapeDtypeStruct((NPAT * 16, 8), _f32),
        compiler_params=pltpu.CompilerParams(
            dimension_semantics=("arbitrary",),
            vmem_limit_bytes=100 * 1024 * 1024),
    )(startP, w1, b1, w2, b2, w3, b3, w4, b4)


def _k2_body(F_hbm, g1_hbm, out1_hbm, g1_v, buf1, sems):
    wid = lax.axis_index("c") * NS + lax.axis_index("s")
    base_p = wid * PPT

    def big(grp, _):
        pltpu.sync_copy(g1_hbm.at[pl.ds(base_p + grp * 8, 8)], g1_v)
        hs = []
        for k in range(8):
            hs.append(pltpu.async_copy(F_hbm.at[g1_v.at[k]], buf1.at[k],
                                       sems.at[k]))
        ws = []
        for k in range(8):
            hs[k].wait()
            p = grp * 8 + k
            ws.append(pltpu.async_copy(buf1.at[k], out1_hbm.at[base_p + p],
                                       sems.at[k]))
        for w in ws:
            w.wait()
        return 0

    lax.fori_loop(0, PPT // 8, big, 0)


def _sc_gather(F, g1):
    mesh = plsc.VectorSubcoreMesh(core_axis_name="c", subcore_axis_name="s")
    fn = functools.partial(
        pl.kernel,
        out_type=jax.ShapeDtypeStruct((NPAT, 32, 128), _f32),
        mesh=mesh,
        scratch_types=[
            pltpu.VMEM((8, 32), _i32),
            pltpu.VMEM((8, 32, 128), _f32),
            pltpu.SemaphoreType.DMA((8,)),
        ],
        compiler_params=pltpu.CompilerParams(use_tc_tiling_on_sc=False),
    )(_k2_body)
    return fn(F, g1)


def _k4_body(base_hbm, fin_hbm, sb_hbm, out_hbm, cbuf, sidx_v, fin_v, sems):
    wid = lax.axis_index("c") * NS + lax.axis_index("s")
    rbase = wid * 262144
    for ch in range(8):
        pltpu.sync_copy(base_hbm.at[pl.ds(rbase + ch * 32768, 32768)], cbuf)
        pltpu.sync_copy(cbuf, out_hbm.at[pl.ds(rbase + ch * 32768, 32768)])
    plsc.subcore_barrier()
    pltpu.sync_copy(sb_hbm.at[pl.ds(wid * 32, 32)], sidx_v)
    pltpu.sync_copy(fin_hbm.at[pl.ds(wid * 4096, 4096)], fin_v)
    hs = []
    for j in range(32):
        hs.append(pltpu.async_copy(fin_v.at[pl.ds(j * 128, 128)],
                                   out_hbm.at[sidx_v.at[j]], sems))
    for h in hs:
        h.wait()


def _sc_scatter(basep, fin2, sb):
    mesh = plsc.VectorSubcoreMesh(core_axis_name="c", subcore_axis_name="s")
    fn = functools.partial(
        pl.kernel,
        out_type=jax.ShapeDtypeStruct((B * 262144,), _f32),
        mesh=mesh,
        scratch_types=[
            pltpu.VMEM((32768,), _f32),
            pltpu.VMEM((32, 128), _i32),
            pltpu.VMEM((4096,), _f32),
            pltpu.SemaphoreType.DMA,
        ],
        compiler_params=pltpu.CompilerParams(use_tc_tiling_on_sc=False),
    )(_k4_body)
    return fn(basep, fin2, sb)


def kernel(fake_coarse_alpha, fake_coarse_error, fake_coarse_hidden,
           input_tensor, conv1_w, conv1_b, conv2_w, conv2_b, conv3_w, conv3_b,
           conv4_w, conv4_b):
    err1 = fake_coarse_error.reshape(B, HC * WC)
    err2 = fake_coarse_error.reshape(B * HC, WC)
    g1T, sT = _topk_indices(err1, err2)
    g1 = g1T.T
    sb = sT.T.reshape(NPAT // 8, 128)

    ua = jnp.repeat(jnp.repeat(fake_coarse_alpha, 2, axis=2), 2, axis=3)
    uh = jnp.repeat(jnp.repeat(fake_coarse_hidden, 2, axis=2), 2, axis=3)
    dn = input_tensor[:, :, 1::2, 1::2]
    quad = input_tensor.transpose(0, 2, 3, 1).reshape(B, 256, 2, 256, 2, 3)
    quad = quad.transpose(0, 1, 3, 2, 4, 5).reshape(B, 256, 256, 12)
    Fimg = jnp.concatenate([ua, uh, dn], axis=1).transpose(0, 2, 3, 1)
    Fimg = jnp.concatenate(
        [Fimg, quad, jnp.zeros((B, 256, 256, 16), _f32)], axis=-1)
    F = Fimg.reshape(B * 256 * 128, 128)

    startP = _sc_gather(F, g1)
    startP = startP.reshape(NPAT, 8, 8, 64)

    w1m = conv1_w.transpose(2, 3, 1, 0).reshape(324, 32).astype(_bf16)
    w2m = conv2_w.transpose(2, 3, 1, 0).reshape(288, 64).astype(_bf16)
    w3m = conv3_w.transpose(2, 3, 1, 0).reshape(603, 32).astype(_bf16)
    w4m = jnp.pad(conv4_w.transpose(2, 3, 1, 0).reshape(288, 1),
                  ((0, 0), (0, 7))).astype(_bf16)
    b1r = conv1_b.reshape(1, 32)
    b2r = conv2_b.reshape(1, 64)
    b3r = conv3_b.reshape(1, 32)
    b4r = jnp.pad(conv4_b.reshape(1, 1), ((0, 0), (0, 7)))

    finw = _convs(startP, w1m, b1r, w2m, b2r, w3m, b3r, w4m, b4r)
    fin2 = finw[:, 0]

    basep = jnp.repeat(jnp.repeat(fake_coarse_alpha[:, 0], 4, axis=1),
                       4, axis=2).reshape(B * 262144)
    out = _sc_scatter(basep, fin2, sb)
    return out.reshape(B, 1, H, W)
